# baseline (speedup 1.0000x reference)
import functools

import jax
import jax.numpy as jnp
from jax import lax
from jax.experimental import pallas as pl
from jax.experimental.pallas import tpu as pltpu
from jax.experimental.pallas import tpu_sc as plsc

_EMBED_DIM = 2048
_NUM_CORES = 2
_NUM_SUBCORES = 16
_NUM_WORKERS = _NUM_CORES * _NUM_SUBCORES
_UNROLL = 16


def _gather_call(ids_flat, token_embeds, num_tokens):
    bpw = num_tokens // _NUM_WORKERS
    mesh = plsc.VectorSubcoreMesh(core_axis_name="core", subcore_axis_name="subcore")

    @functools.partial(
        pl.kernel,
        out_type=jax.ShapeDtypeStruct((num_tokens, _EMBED_DIM), token_embeds.dtype),
        mesh=mesh,
        scratch_types=[
            pltpu.VMEM((bpw,), jnp.int32),
            pltpu.SemaphoreType.DMA,
        ],
    )
    def gather_kernel(tab_hbm, idx_hbm, out_hbm, idx_v, sem):
        wid = lax.axis_index("subcore") * _NUM_CORES + lax.axis_index("core")
        base = wid * bpw
        pltpu.sync_copy(idx_hbm.at[pl.ds(base, bpw)], idx_v)

        @pl.loop(0, bpw, step=_UNROLL)
        def _(c):
            ids_vec = idx_v[pl.ds(c, _UNROLL)]
            for j in range(_UNROLL):
                row = ids_vec[j]
                pltpu.make_async_copy(
                    tab_hbm.at[pl.ds(row, 1)],
                    out_hbm.at[pl.ds(base + c + j, 1)],
                    sem,
                ).start()

        pltpu.make_async_copy(
            tab_hbm.at[pl.ds(0, bpw)], out_hbm.at[pl.ds(base, bpw)], sem
        ).wait()

    return gather_kernel(token_embeds, ids_flat)


def kernel(input_ids, token_embeds):
    batch, seq_len = input_ids.shape
    num_tokens = batch * seq_len
    ids_flat = input_ids.astype(jnp.int32).reshape(num_tokens)
    out = _gather_call(ids_flat, token_embeds, num_tokens)
    return out.reshape(batch, seq_len, token_embeds.shape[1])


# speedup vs baseline: 1.0048x; 1.0048x over previous
import functools

import jax
import jax.numpy as jnp
from jax.experimental import pallas as pl
from jax.experimental.pallas import tpu as pltpu

_EMBED_DIM = 2048
_K = 16


def _tc_gather_call(ids_flat, token_embeds, num_tokens):
    def body(idx_s, tab_ref, out_ref, *sems):
        def row_cp(i, j):
            return pltpu.make_async_copy(
                tab_ref.at[pl.ds(idx_s[i], 1)],
                out_ref.at[pl.ds(i, 1)],
                sems[j],
            )

        for j in range(_K):
            row_cp(j, j).start()

        @pl.loop(_K, num_tokens, step=_K)
        def _(c):
            for j in range(_K):
                row_cp(c - _K + j, j).wait()
                row_cp(c + j, j).start()

        for j in range(_K):
            row_cp(num_tokens - _K + j, j).wait()

    return pl.pallas_call(
        body,
        out_shape=jax.ShapeDtypeStruct((num_tokens, _EMBED_DIM), token_embeds.dtype),
        in_specs=[
            pl.BlockSpec(memory_space=pltpu.MemorySpace.SMEM),
            pl.BlockSpec(memory_space=pltpu.MemorySpace.HBM),
        ],
        out_specs=pl.BlockSpec(memory_space=pltpu.MemorySpace.HBM),
        scratch_shapes=[pltpu.SemaphoreType.DMA for _ in range(_K)],
    )(ids_flat, token_embeds)


def kernel(input_ids, token_embeds):
    batch, seq_len = input_ids.shape
    num_tokens = batch * seq_len
    ids_flat = input_ids.astype(jnp.int32).reshape(num_tokens)
    out = _tc_gather_call(ids_flat, token_embeds, num_tokens)
    return out.reshape(batch, seq_len, token_embeds.shape[1])


# speedup vs baseline: 35.1132x; 34.9456x over previous
import functools

import jax
import jax.numpy as jnp
from jax import lax
from jax.experimental import pallas as pl
from jax.experimental.pallas import tpu as pltpu
from jax.experimental.pallas import tpu_sc as plsc

_EMBED_DIM = 2048
_NUM_CORES = 2
_NUM_SUBCORES = 16
_NUM_WORKERS = _NUM_CORES * _NUM_SUBCORES
_CHUNK = 8
_NBUF = 4


def _gather_call(ids_flat, token_embeds, num_tokens):
    bpw = num_tokens // _NUM_WORKERS
    nch = bpw // _CHUNK
    assert nch % _NBUF == 0
    mesh = plsc.VectorSubcoreMesh(core_axis_name="core", subcore_axis_name="subcore")

    @functools.partial(
        pl.kernel,
        out_type=jax.ShapeDtypeStruct((num_tokens, _EMBED_DIM), token_embeds.dtype),
        mesh=mesh,
        scratch_types=(
            [pltpu.VMEM((bpw,), jnp.int32)]
            + [pltpu.VMEM((_CHUNK, _EMBED_DIM), jnp.float32) for _ in range(_NBUF)]
            + [pltpu.SemaphoreType.DMA for _ in range(2 * _NBUF)]
        ),
    )
    def gather_kernel(tab_hbm, idx_hbm, out_hbm, idx_v, *scratch):
        bufs = scratch[:_NBUF]
        gsems = scratch[_NBUF : 2 * _NBUF]
        osems = scratch[2 * _NBUF :]
        wid = lax.axis_index("core") * _NUM_SUBCORES + lax.axis_index("subcore")
        base = wid * bpw
        pltpu.sync_copy(idx_hbm.at[pl.ds(base, bpw)], idx_v)

        def gather_cp(c, b):
            return pltpu.make_async_copy(
                tab_hbm.at[idx_v.at[pl.ds(c * _CHUNK, _CHUNK)]], bufs[b], gsems[b]
            )

        def out_cp(c, b):
            return pltpu.make_async_copy(
                bufs[b], out_hbm.at[pl.ds(base + c * _CHUNK, _CHUNK)], osems[b]
            )

        for b in range(_NBUF):
            gather_cp(b, b).start()

        @pl.loop(0, nch - _NBUF, step=_NBUF)
        def _(c):
            for b in range(_NBUF):
                gather_cp(c + b, b).wait()
                out_cp(c + b, b).start()
            for b in range(_NBUF):
                out_cp(c + b, b).wait()
                gather_cp(c + _NBUF + b, b).start()

        for b in range(_NBUF):
            gather_cp(nch - _NBUF + b, b).wait()
            out_cp(nch - _NBUF + b, b).start()
        for b in range(_NBUF):
            out_cp(nch - _NBUF + b, b).wait()

    return gather_kernel(token_embeds, ids_flat)


def kernel(input_ids, token_embeds):
    batch, seq_len = input_ids.shape
    num_tokens = batch * seq_len
    ids_flat = input_ids.astype(jnp.int32).reshape(num_tokens)
    out = _gather_call(ids_flat, token_embeds, num_tokens)
    return out.reshape(batch, seq_len, token_embeds.shape[1])


# speedup vs baseline: 35.3780x; 1.0075x over previous
import functools

import jax
import jax.numpy as jnp
from jax import lax
from jax.experimental import pallas as pl
from jax.experimental.pallas import tpu as pltpu
from jax.experimental.pallas import tpu_sc as plsc

_NUM_CORES = 2
_NUM_SUBCORES = 16
_NUM_WORKERS = _NUM_CORES * _NUM_SUBCORES
_CHUNK = 8
_NBUF = 4


def _gather_call(input_ids, token_embeds):
    batch, seq_len = input_ids.shape
    embed_dim = token_embeds.shape[1]
    num_tokens = batch * seq_len
    bpw = num_tokens // _NUM_WORKERS
    wpb = seq_len // bpw
    nch = bpw // _CHUNK
    assert nch % _NBUF == 0 and _CHUNK % 8 == 0
    mesh = plsc.VectorSubcoreMesh(core_axis_name="core", subcore_axis_name="subcore")

    @functools.partial(
        pl.kernel,
        out_type=jax.ShapeDtypeStruct((batch, seq_len, embed_dim), token_embeds.dtype),
        mesh=mesh,
        scratch_types=(
            [pltpu.VMEM((bpw,), jnp.int32)]
            + [pltpu.VMEM((_CHUNK, embed_dim), jnp.float32) for _ in range(_NBUF)]
            + [pltpu.SemaphoreType.DMA for _ in range(2 * _NBUF)]
        ),
    )
    def gather_kernel(tab_hbm, idx_hbm, out_hbm, idx_v, *scratch):
        bufs = scratch[:_NBUF]
        gsems = scratch[_NBUF : 2 * _NBUF]
        osems = scratch[2 * _NBUF :]
        wid = lax.axis_index("core") * _NUM_SUBCORES + lax.axis_index("subcore")
        b_row = wid // wpb
        s_base = (wid % wpb) * bpw
        pltpu.sync_copy(idx_hbm.at[b_row, pl.ds(s_base, bpw)], idx_v)

        def gather_cp(c, b):
            return pltpu.make_async_copy(
                tab_hbm.at[idx_v.at[pl.ds(c * _CHUNK, _CHUNK)]], bufs[b], gsems[b]
            )

        def out_cp(c, b):
            return pltpu.make_async_copy(
                bufs[b],
                out_hbm.at[b_row, pl.ds(s_base + c * _CHUNK, _CHUNK)],
                osems[b],
            )

        for b in range(_NBUF):
            gather_cp(b, b).start()

        @pl.loop(0, nch - _NBUF, step=_NBUF)
        def _(c):
            for b in range(_NBUF):
                gather_cp(c + b, b).wait()
                out_cp(c + b, b).start()
            for b in range(_NBUF):
                out_cp(c + b, b).wait()
                gather_cp(c + _NBUF + b, b).start()

        for b in range(_NBUF):
            gather_cp(nch - _NBUF + b, b).wait()
            out_cp(nch - _NBUF + b, b).start()
        for b in range(_NBUF):
            out_cp(nch - _NBUF + b, b).wait()

    return gather_kernel(token_embeds, input_ids)


def kernel(input_ids, token_embeds):
    return _gather_call(input_ids.astype(jnp.int32), token_embeds)
